# Initial kernel scaffold; baseline (speedup 1.0000x reference)
import jax
import jax.numpy as jnp
from jax.experimental import pallas as pl


def kernel(embed, time, embeddings):
    raise NotImplementedError("write your pallas kernel here")


# speedup vs baseline: 1.9835x; 1.9835x over previous
import functools

import jax
import jax.numpy as jnp
from jax import lax
from jax.experimental import pallas as pl
from jax.experimental.pallas import tpu as pltpu
from jax.experimental.pallas import tpu_sc as plsc

D_EMBED = 128
MAX_LEN = 2048

NC = 2
NS = 16
NW = NC * NS
LANES = 16

CHUNK = 128


def _sc_body(embed_hbm, time_hbm, table_hbm, out_hbm,
             idx_v, g_v, e_v, gsem, esem):
    wid = lax.axis_index("s") * NC + lax.axis_index("c")
    n_rows = embed_hbm.shape[0]
    rows_per_w = n_rows // NW
    n_chunks = rows_per_w // CHUNK
    base = wid * rows_per_w

    pltpu.sync_copy(time_hbm.at[pl.ds(base, rows_per_w)], idx_v)

    def chunk_body(c, _):
        row0 = base + c * CHUNK
        gcp = pltpu.async_copy(
            table_hbm.at[idx_v.at[pl.ds(c * CHUNK, CHUNK)]], g_v, gsem)
        ecp = pltpu.async_copy(embed_hbm.at[pl.ds(row0, CHUNK)], e_v, esem)
        gcp.wait()
        ecp.wait()

        def row_body(j, _):
            for k in range(D_EMBED // LANES):
                s = pl.ds(k * LANES, LANES)
                g_v[j, s] = g_v[j, s] + e_v[j, s]
            return ()

        lax.fori_loop(0, CHUNK, row_body, (), unroll=2)
        pltpu.sync_copy(g_v, out_hbm.at[pl.ds(row0, CHUNK)])
        return ()

    lax.fori_loop(0, n_chunks, chunk_body, ())


@jax.jit
def _temporal_encoding_sc(embed_flat, time1d, table):
    n_rows = embed_flat.shape[0]
    mesh = plsc.VectorSubcoreMesh(core_axis_name="c", subcore_axis_name="s")
    return pl.kernel(
        _sc_body,
        out_type=jax.ShapeDtypeStruct((n_rows, D_EMBED), jnp.float32),
        mesh=mesh,
        scratch_types=[
            pltpu.VMEM((n_rows // NW,), jnp.int32),
            pltpu.VMEM((CHUNK, D_EMBED), jnp.float32),
            pltpu.VMEM((CHUNK, D_EMBED), jnp.float32),
            pltpu.SemaphoreType.DMA,
            pltpu.SemaphoreType.DMA,
        ],
        name="temporal_encoding_sc",
    )(embed_flat, time1d, table)


def kernel(embed, time, embeddings):
    b, t, d = embed.shape
    n_rows = b * t
    embed_flat = embed.reshape(n_rows, d)
    time1d = time.astype(jnp.int32).reshape(n_rows)
    out = _temporal_encoding_sc(embed_flat, time1d, embeddings)
    return out.reshape(b, t, d)


# speedup vs baseline: 5.1905x; 2.6169x over previous
import functools

import jax
import jax.numpy as jnp
from jax import lax
from jax.experimental import pallas as pl
from jax.experimental.pallas import tpu as pltpu
from jax.experimental.pallas import tpu_sc as plsc

D_EMBED = 128
MAX_LEN = 2048

NC = 2
NS = 16
NW = NC * NS

CHUNK = 128
NBUF = 5


def _sc_body(embed_hbm, time_hbm, table_hbm, out_hbm, idx_v, buf_v, *sems):
    esem = sems[:NBUF]
    gsem = sems[NBUF:2 * NBUF]
    osem = sems[2 * NBUF:]

    wid = lax.axis_index("s") * NC + lax.axis_index("c")
    n_rows = embed_hbm.shape[0]
    rows_per_w = n_rows // NW
    n_chunks = rows_per_w // CHUNK
    n_groups = n_chunks // NBUF
    base = wid * rows_per_w

    pltpu.sync_copy(time_hbm.at[pl.ds(base, rows_per_w)], idx_v)

    def slot(b):
        return buf_v.at[pl.ds(b * CHUNK, CHUNK)]

    for b in range(NBUF):
        pltpu.async_copy(embed_hbm.at[pl.ds(base + b * CHUNK, CHUNK)],
                         slot(b), esem[b])

    def group_body(g, _):
        row_g = base + g * (NBUF * CHUNK)
        off_g = g * (NBUF * CHUNK)

        gathers = []
        for b in range(NBUF):
            pltpu.make_async_copy(
                embed_hbm.at[pl.ds(row_g + b * CHUNK, CHUNK)],
                slot(b), esem[b]).wait()
            gathers.append(pltpu.async_copy(
                table_hbm.at[idx_v.at[pl.ds(off_g + b * CHUNK, CHUNK)]],
                slot(b), gsem[b], add=True))

        stores = []
        for b in range(NBUF):
            gathers[b].wait()
            stores.append(pltpu.async_copy(
                slot(b), out_hbm.at[pl.ds(row_g + b * CHUNK, CHUNK)],
                osem[b]))

        for b in range(NBUF):
            stores[b].wait()

            @pl.when(g + 1 < n_groups)
            def _():
                pltpu.async_copy(
                    embed_hbm.at[
                        pl.ds(row_g + (NBUF + b) * CHUNK, CHUNK)],
                    slot(b), esem[b])

        return ()

    lax.fori_loop(0, n_groups, group_body, ())


@jax.jit
def _temporal_encoding_sc(embed_flat, time1d, table):
    n_rows = embed_flat.shape[0]
    mesh = plsc.VectorSubcoreMesh(core_axis_name="c", subcore_axis_name="s")
    return pl.kernel(
        _sc_body,
        out_type=jax.ShapeDtypeStruct((n_rows, D_EMBED), jnp.float32),
        mesh=mesh,
        scratch_types=[
            pltpu.VMEM((n_rows // NW,), jnp.int32),
            pltpu.VMEM((NBUF * CHUNK, D_EMBED), jnp.float32),
        ] + [pltpu.SemaphoreType.DMA] * (3 * NBUF),
        name="temporal_encoding_sc",
    )(embed_flat, time1d, table)


def kernel(embed, time, embeddings):
    b, t, d = embed.shape
    n_rows = b * t
    embed_flat = embed.reshape(n_rows, d)
    time1d = time.astype(jnp.int32).reshape(n_rows)
    out = _temporal_encoding_sc(embed_flat, time1d, embeddings)
    return out.reshape(b, t, d)


# speedup vs baseline: 7.4301x; 1.4315x over previous
import functools

import jax
import jax.numpy as jnp
from jax import lax
from jax.experimental import pallas as pl
from jax.experimental.pallas import tpu as pltpu
from jax.experimental.pallas import tpu_sc as plsc

D_EMBED = 128
MAX_LEN = 2048

NC = 2
NS = 16
NW = NC * NS

CHUNK = 128
NBUF = 5


def _sc_body(embed_hbm, time_hbm, table_hbm, out_hbm, idx_v, buf_v,
             table_sp, *sems):
    esem = sems[:NBUF]
    gsem = sems[NBUF:2 * NBUF]
    osem = sems[2 * NBUF:]

    sid = lax.axis_index("s")
    wid = sid * NC + lax.axis_index("c")
    n_rows = embed_hbm.shape[0]
    rows_per_w = n_rows // NW
    n_chunks = rows_per_w // CHUNK
    n_groups = n_chunks // NBUF
    base = wid * rows_per_w

    def slot(b):
        return buf_v.at[pl.ds(b * CHUNK, CHUNK)]

    for b in range(NBUF):
        pltpu.async_copy(embed_hbm.at[pl.ds(base + b * CHUNK, CHUNK)],
                         slot(b), esem[b])

    t_rows = MAX_LEN // NS
    pltpu.sync_copy(table_hbm.at[pl.ds(sid * t_rows, t_rows)],
                    table_sp.at[pl.ds(sid * t_rows, t_rows)])
    plsc.subcore_barrier()

    pltpu.sync_copy(time_hbm.at[pl.ds(base, rows_per_w)], idx_v)

    def group_body(g, _):
        row_g = base + g * (NBUF * CHUNK)
        off_g = g * (NBUF * CHUNK)

        gathers = []
        for b in range(NBUF):
            pltpu.make_async_copy(
                embed_hbm.at[pl.ds(row_g + b * CHUNK, CHUNK)],
                slot(b), esem[b]).wait()
            gathers.append(pltpu.async_copy(
                table_sp.at[idx_v.at[pl.ds(off_g + b * CHUNK, CHUNK)]],
                slot(b), gsem[b], add=True))

        stores = []
        for b in range(NBUF):
            gathers[b].wait()
            stores.append(pltpu.async_copy(
                slot(b), out_hbm.at[pl.ds(row_g + b * CHUNK, CHUNK)],
                osem[b]))

        for b in range(NBUF):
            stores[b].wait()

            @pl.when(g + 1 < n_groups)
            def _():
                pltpu.async_copy(
                    embed_hbm.at[
                        pl.ds(row_g + (NBUF + b) * CHUNK, CHUNK)],
                    slot(b), esem[b])

        return ()

    lax.fori_loop(0, n_groups, group_body, ())


@jax.jit
def _temporal_encoding_sc(embed_flat, time1d, table):
    n_rows = embed_flat.shape[0]
    mesh = plsc.VectorSubcoreMesh(core_axis_name="c", subcore_axis_name="s")
    return pl.kernel(
        _sc_body,
        out_type=jax.ShapeDtypeStruct((n_rows, D_EMBED), jnp.float32),
        mesh=mesh,
        scratch_types=[
            pltpu.VMEM((n_rows // NW,), jnp.int32),
            pltpu.VMEM((NBUF * CHUNK, D_EMBED), jnp.float32),
            pltpu.VMEM_SHARED((MAX_LEN, D_EMBED), jnp.float32),
        ] + [pltpu.SemaphoreType.DMA] * (3 * NBUF),
        name="temporal_encoding_sc",
    )(embed_flat, time1d, table)


def kernel(embed, time, embeddings):
    b, t, d = embed.shape
    n_rows = b * t
    embed_flat = embed.reshape(n_rows, d)
    time1d = time.astype(jnp.int32).reshape(n_rows)
    out = _temporal_encoding_sc(embed_flat, time1d, embeddings)
    return out.reshape(b, t, d)
